# baseline (speedup 1.0000x reference)
import jax
import jax.numpy as jnp
from jax.experimental import pallas as pl
from jax.experimental.pallas import tpu as pltpu


_ROWS = 8192
_COLS = 1024
_N_CHUNKS = 8
_CHUNK = _ROWS // _N_CHUNKS


def _copy_kernel(x_hbm, o_hbm, sems):
    for i in range(_N_CHUNKS):
        pltpu.make_async_copy(
            x_hbm.at[pl.ds(i * _CHUNK, _CHUNK), :],
            o_hbm.at[pl.ds(i * _CHUNK, _CHUNK), :],
            sems.at[i],
        ).start()
    for i in range(_N_CHUNKS):
        pltpu.make_async_copy(
            x_hbm.at[pl.ds(i * _CHUNK, _CHUNK), :],
            o_hbm.at[pl.ds(i * _CHUNK, _CHUNK), :],
            sems.at[i],
        ).wait()


def kernel(x):
    gathered = pl.pallas_call(
        _copy_kernel,
        in_specs=[pl.BlockSpec(memory_space=pl.ANY)],
        out_specs=pl.BlockSpec(memory_space=pl.ANY),
        out_shape=jax.ShapeDtypeStruct((_ROWS, _COLS), x.dtype),
        scratch_shapes=[pltpu.SemaphoreType.DMA((_N_CHUNKS,))],
    )(x)
    sizes = jnp.array([_ROWS], dtype=jnp.int32)
    return (gathered, sizes)


# speedup vs baseline: 44.6845x; 44.6845x over previous
import jax
import jax.numpy as jnp
from jax.experimental import pallas as pl
from jax.experimental.pallas import tpu as pltpu


_ROWS = 8192
_COLS = 1024
_BLOCK_ROWS = 2048


def _copy_kernel(x_ref, o_ref):
    o_ref[...] = x_ref[...]


def kernel(x):
    n_blocks = _ROWS // _BLOCK_ROWS
    gathered = pl.pallas_call(
        _copy_kernel,
        grid=(n_blocks,),
        in_specs=[pl.BlockSpec((_BLOCK_ROWS, _COLS), lambda i: (i, 0))],
        out_specs=pl.BlockSpec((_BLOCK_ROWS, _COLS), lambda i: (i, 0)),
        out_shape=jax.ShapeDtypeStruct((_ROWS, _COLS), x.dtype),
        compiler_params=pltpu.CompilerParams(
            dimension_semantics=("parallel",),
        ),
    )(x)
    sizes = jnp.array([_ROWS], dtype=jnp.int32)
    return (gathered, sizes)
